# baseline (speedup 1.0000x reference)
import functools

import jax
import jax.numpy as jnp
from jax import lax
from jax.experimental import pallas as pl
from jax.experimental.pallas import tpu as pltpu
from jax.experimental.pallas import tpu_sc as plsc

KMAX = 10
KMIN = 10

NUM_CORES = 2
NUM_SUBCORES = 16
LANES = 16
TILE = 16
GROUP = 16


def _batcher(num):
    def oe_merge(lo, nn, r):
        step = r * 2
        if step < nn:
            yield from oe_merge(lo, nn, step)
            yield from oe_merge(lo + r, nn, step)
            for i in range(lo + r, lo + nn - r, step):
                yield (i, i + r)
        else:
            yield (lo, lo + r)

    def srt(lo, nn):
        if nn > 1:
            m = nn // 2
            yield from srt(lo, m)
            yield from srt(lo + m, m)
            yield from oe_merge(lo, nn, 1)

    return list(srt(0, num))


_SORT16 = _batcher(GROUP)


def _sort16_desc(v):
    v = list(v)
    for i, j in _SORT16:
        hi = jnp.maximum(v[i], v[j])
        lo = jnp.minimum(v[i], v[j])
        v[i], v[j] = hi, lo
    return v


def _merge_top(T, A):
    C = [jnp.maximum(T[i], A[15 - i]) for i in range(16)]
    for d in (8, 4, 2, 1):
        for j in range(16):
            if (j % (2 * d)) < d:
                hi = jnp.maximum(C[j], C[j + d])
                lo = jnp.minimum(C[j], C[j + d])
                C[j], C[j + d] = hi, lo
    return C


def _merge_bot(B, A):
    C = [jnp.minimum(B[i], A[i]) for i in range(16)]
    for d in (8, 4, 2, 1):
        for j in range(16):
            if (j % (2 * d)) < d:
                lo = jnp.minimum(C[j], C[j + d])
                hi = jnp.maximum(C[j], C[j + d])
                C[j], C[j + d] = lo, hi
    return C


def _make_sc_kernel(rows, n, num_cores=NUM_CORES):
    num_workers = num_cores * NUM_SUBCORES
    rows_per_w = rows // num_workers
    tiles = rows_per_w // TILE
    groups = n // GROUP

    mesh = plsc.VectorSubcoreMesh(
        core_axis_name="c", subcore_axis_name="s",
        num_cores=num_cores, num_subcores=NUM_SUBCORES)

    @functools.partial(
        pl.kernel,
        mesh=mesh,
        out_type=jax.ShapeDtypeStruct((rows,), jnp.float32),
        scratch_types=[
            pltpu.VMEM((TILE * n,), jnp.float32),
            pltpu.VMEM((rows_per_w,), jnp.float32),
        ],
        compiler_params=pltpu.CompilerParams(
            use_tc_tiling_on_sc=False, needs_layout_passes=False),
    )
    def k(x_hbm, out_hbm, buf_v, out_v):
        wid = lax.axis_index("s") * NUM_CORES + lax.axis_index("c")
        row0 = wid * rows_per_w
        lanes = lax.iota(jnp.int32, LANES)
        lanebase = lanes * n
        rot = lanes * 17
        neg = jnp.full((LANES,), -jnp.inf, jnp.float32)
        pos = jnp.full((LANES,), jnp.inf, jnp.float32)

        def tile_body(t, _):
            pltpu.sync_copy(x_hbm.at[pl.ds((row0 + t * TILE) * n, TILE * n)],
                            buf_v)

            def group_body(g, carry):
                T = list(carry[:16])
                Bo = list(carry[16:32])
                iv = carry[32]
                A = [plsc.load_gather(
                        buf_v, [lanebase + ((iv + kk) & (n - 1))])
                     for kk in range(GROUP)]
                A = _sort16_desc(A)
                T = _merge_top(T, A)
                Bo = _merge_bot(Bo, A)
                return tuple(T) + tuple(Bo) + (iv + GROUP,)

            init = (neg,) * 16 + (pos,) * 16 + (rot,)
            fin = lax.fori_loop(0, groups, group_body, init)
            top_sum = fin[0]
            for j in range(1, KMAX):
                top_sum = top_sum + fin[j]
            bot_sum = fin[16]
            for j in range(1, KMIN):
                bot_sum = bot_sum + fin[16 + j]
            res = (top_sum / KMAX + bot_sum / KMIN) * jnp.float32(0.5)
            out_v[pl.ds(t * TILE, TILE)] = res
            return 0

        lax.fori_loop(0, tiles, tile_body, 0)
        pltpu.sync_copy(out_v, out_hbm.at[pl.ds(row0, rows_per_w)])

    return k


ROWS_PER_BLOCK = 256


def _tc_body(x_ref, o_ref):
    x = x_ref[...]
    R, N = x.shape
    col = lax.broadcasted_iota(jnp.int32, (R, N), 1)
    neg = jnp.float32(-jnp.inf)
    pos = jnp.float32(jnp.inf)

    xt = x
    top_sum = jnp.zeros((R, 1), jnp.float32)
    for _ in range(KMAX):
        m = jnp.max(xt, axis=1, keepdims=True)
        top_sum = top_sum + m
        idx = jnp.min(jnp.where(xt == m, col, N), axis=1, keepdims=True)
        xt = jnp.where(col == idx, neg, xt)

    xb = x
    bot_sum = jnp.zeros((R, 1), jnp.float32)
    for _ in range(KMIN):
        m = jnp.min(xb, axis=1, keepdims=True)
        bot_sum = bot_sum + m
        idx = jnp.min(jnp.where(xb == m, col, N), axis=1, keepdims=True)
        xb = jnp.where(col == idx, pos, xb)

    o_ref[...] = (top_sum / KMAX + bot_sum / KMIN) * 0.5


def _tc_kernel(x, n):
    rows = x.shape[0]
    R = ROWS_PER_BLOCK
    out = pl.pallas_call(
        _tc_body,
        grid=(rows // R,),
        in_specs=[pl.BlockSpec((R, n), lambda i: (i, 0))],
        out_specs=pl.BlockSpec((R, 1), lambda i: (i, 0)),
        out_shape=jax.ShapeDtypeStruct((rows, 1), jnp.float32),
    )(x)
    return out.reshape(rows)


def kernel(input):
    B, C, H, W = input.shape
    n = H * W
    rows = B * C
    half = rows // 2
    xf = input.reshape(rows * n)
    k1 = _make_sc_kernel(half, n, num_cores=1)
    out_a = k1(xf[: half * n])
    out_b = k1(xf[half * n:])
    out = jnp.concatenate([out_a, out_b])
    return out.reshape(B, C)


# speedup vs baseline: 1.3921x; 1.3921x over previous
import functools

import jax
import jax.numpy as jnp
from jax import lax
from jax.experimental import pallas as pl
from jax.experimental.pallas import tpu as pltpu
from jax.experimental.pallas import tpu_sc as plsc

KMAX = 10
KMIN = 10

NUM_CORES = 2
NUM_SUBCORES = 16
LANES = 16
TILE = 16
GROUP = 16


def _batcher(num):
    def oe_merge(lo, nn, r):
        step = r * 2
        if step < nn:
            yield from oe_merge(lo, nn, step)
            yield from oe_merge(lo + r, nn, step)
            for i in range(lo + r, lo + nn - r, step):
                yield (i, i + r)
        else:
            yield (lo, lo + r)

    def srt(lo, nn):
        if nn > 1:
            m = nn // 2
            yield from srt(lo, m)
            yield from srt(lo + m, m)
            yield from oe_merge(lo, nn, 1)

    return list(srt(0, num))


_SORT16 = _batcher(GROUP)


def _sort16_desc(v):
    v = list(v)
    for i, j in _SORT16:
        hi = jnp.maximum(v[i], v[j])
        lo = jnp.minimum(v[i], v[j])
        v[i], v[j] = hi, lo
    return v


def _pruned_merge16_ces():
    ces = []
    for d in (8, 4, 2, 1):
        for j in range(16):
            if (j % (2 * d)) < d:
                ces.append((j, j + d))
    needed = set(range(10))
    keep = []
    for i, j in reversed(ces):
        if i in needed or j in needed:
            keep.append((i, j))
            needed.add(i)
            needed.add(j)
    keep.reverse()
    return keep


_MERGE16 = _pruned_merge16_ces()


def _merge_top(T, A):
    D = list(T[:6]) + [jnp.maximum(T[6], A[9]), jnp.maximum(T[7], A[8]),
                       jnp.maximum(T[8], A[7]), jnp.maximum(T[9], A[6]),
                       A[5], A[4], A[3], A[2], A[1], A[0]]
    for i, j in _MERGE16:
        hi = jnp.maximum(D[i], D[j])
        lo = jnp.minimum(D[i], D[j])
        D[i], D[j] = hi, lo
    return D[:10]


def _merge_bot(B, A):
    D = list(B[:6]) + [jnp.minimum(B[6], A[6]), jnp.minimum(B[7], A[7]),
                       jnp.minimum(B[8], A[8]), jnp.minimum(B[9], A[9]),
                       A[10], A[11], A[12], A[13], A[14], A[15]]
    for i, j in _MERGE16:
        lo = jnp.minimum(D[i], D[j])
        hi = jnp.maximum(D[i], D[j])
        D[i], D[j] = lo, hi
    return D[:10]


def _make_sc_kernel(rows, n, num_cores=NUM_CORES):
    num_workers = num_cores * NUM_SUBCORES
    rows_per_w = rows // num_workers
    tiles = rows_per_w // TILE
    groups = n // GROUP

    mesh = plsc.VectorSubcoreMesh(
        core_axis_name="c", subcore_axis_name="s",
        num_cores=num_cores, num_subcores=NUM_SUBCORES)

    @functools.partial(
        pl.kernel,
        mesh=mesh,
        out_type=jax.ShapeDtypeStruct((rows,), jnp.float32),
        scratch_types=[
            pltpu.VMEM((TILE * n,), jnp.float32),
            pltpu.VMEM((rows_per_w,), jnp.float32),
        ],
        compiler_params=pltpu.CompilerParams(
            use_tc_tiling_on_sc=False, needs_layout_passes=False),
    )
    def k(x_hbm, out_hbm, buf_v, out_v):
        wid = lax.axis_index("s") * NUM_CORES + lax.axis_index("c")
        row0 = wid * rows_per_w
        lanes = lax.iota(jnp.int32, LANES)
        lanebase = lanes * n
        rot = lanes * 17
        neg = jnp.full((LANES,), -jnp.inf, jnp.float32)
        pos = jnp.full((LANES,), jnp.inf, jnp.float32)

        SPLIT = (n - 255 - (GROUP - 1)) // GROUP

        def tile_body(t, _):
            pltpu.sync_copy(x_hbm.at[pl.ds((row0 + t * TILE) * n, TILE * n)],
                            buf_v)

            def consume(T, Bo, A):
                A = _sort16_desc(A)
                return _merge_top(T, A), _merge_bot(Bo, A)

            def group_body_a(g, carry):
                T = list(carry[:KMAX])
                Bo = list(carry[KMAX:2 * KMAX])
                iv = carry[2 * KMAX]
                A = [plsc.load_gather(buf_v, [iv + kk])
                     for kk in range(GROUP)]
                T, Bo = consume(T, Bo, A)
                return tuple(T) + tuple(Bo) + (iv + GROUP,)

            def group_body_b(g, carry):
                T = list(carry[:KMAX])
                Bo = list(carry[KMAX:2 * KMAX])
                iv = carry[2 * KMAX]
                A = [plsc.load_gather(
                        buf_v, [lanebase + ((iv + kk) & (n - 1))])
                     for kk in range(GROUP)]
                T, Bo = consume(T, Bo, A)
                return tuple(T) + tuple(Bo) + (iv + GROUP,)

            init = (neg,) * KMAX + (pos,) * KMAX + (lanebase + rot,)
            mid = lax.fori_loop(0, SPLIT, group_body_a, init)
            init_b = mid[:2 * KMAX] + (rot + SPLIT * GROUP,)
            fin = lax.fori_loop(SPLIT, groups, group_body_b, init_b)
            top_sum = fin[0]
            for j in range(1, KMAX):
                top_sum = top_sum + fin[j]
            bot_sum = fin[KMAX]
            for j in range(1, KMIN):
                bot_sum = bot_sum + fin[KMAX + j]
            res = (top_sum / KMAX + bot_sum / KMIN) * jnp.float32(0.5)
            out_v[pl.ds(t * TILE, TILE)] = res
            return 0

        lax.fori_loop(0, tiles, tile_body, 0)
        pltpu.sync_copy(out_v, out_hbm.at[pl.ds(row0, rows_per_w)])

    return k


ROWS_PER_BLOCK = 256


def _tc_body(x_ref, o_ref):
    x = x_ref[...]
    R, N = x.shape
    col = lax.broadcasted_iota(jnp.int32, (R, N), 1)
    neg = jnp.float32(-jnp.inf)
    pos = jnp.float32(jnp.inf)

    xt = x
    top_sum = jnp.zeros((R, 1), jnp.float32)
    for _ in range(KMAX):
        m = jnp.max(xt, axis=1, keepdims=True)
        top_sum = top_sum + m
        idx = jnp.min(jnp.where(xt == m, col, N), axis=1, keepdims=True)
        xt = jnp.where(col == idx, neg, xt)

    xb = x
    bot_sum = jnp.zeros((R, 1), jnp.float32)
    for _ in range(KMIN):
        m = jnp.min(xb, axis=1, keepdims=True)
        bot_sum = bot_sum + m
        idx = jnp.min(jnp.where(xb == m, col, N), axis=1, keepdims=True)
        xb = jnp.where(col == idx, pos, xb)

    o_ref[...] = (top_sum / KMAX + bot_sum / KMIN) * 0.5


def _tc_kernel(x, n):
    rows = x.shape[0]
    R = ROWS_PER_BLOCK
    out = pl.pallas_call(
        _tc_body,
        grid=(rows // R,),
        in_specs=[pl.BlockSpec((R, n), lambda i: (i, 0))],
        out_specs=pl.BlockSpec((R, 1), lambda i: (i, 0)),
        out_shape=jax.ShapeDtypeStruct((rows, 1), jnp.float32),
    )(x)
    return out.reshape(rows)


def kernel(input):
    B, C, H, W = input.shape
    n = H * W
    rows = B * C
    x = input.reshape(rows * n)
    out = _make_sc_kernel(rows, n)(x)
    return out.reshape(B, C)


# speedup vs baseline: 1.5466x; 1.1110x over previous
import functools

import jax
import jax.numpy as jnp
from jax import lax
from jax.experimental import pallas as pl
from jax.experimental.pallas import tpu as pltpu
from jax.experimental.pallas import tpu_sc as plsc

KMAX = 10
KMIN = 10

NUM_CORES = 2
NUM_SUBCORES = 16
LANES = 16
TILE = 16
GROUP = 16


def _batcher(num):
    def oe_merge(lo, nn, r):
        step = r * 2
        if step < nn:
            yield from oe_merge(lo, nn, step)
            yield from oe_merge(lo + r, nn, step)
            for i in range(lo + r, lo + nn - r, step):
                yield (i, i + r)
        else:
            yield (lo, lo + r)

    def srt(lo, nn):
        if nn > 1:
            m = nn // 2
            yield from srt(lo, m)
            yield from srt(lo + m, m)
            yield from oe_merge(lo, nn, 1)

    return list(srt(0, num))


_SORT16 = _batcher(GROUP)


def _sort16_desc(v):
    v = list(v)
    for i, j in _SORT16:
        hi = jnp.maximum(v[i], v[j])
        lo = jnp.minimum(v[i], v[j])
        v[i], v[j] = hi, lo
    return v


def _pruned_merge16_ces():
    ces = []
    for d in (8, 4, 2, 1):
        for j in range(16):
            if (j % (2 * d)) < d:
                ces.append((j, j + d))
    needed = set(range(10))
    keep = []
    for i, j in reversed(ces):
        if i in needed or j in needed:
            keep.append((i, j))
            needed.add(i)
            needed.add(j)
    keep.reverse()
    return keep


_MERGE16 = _pruned_merge16_ces()


def _merge_top(T, A):
    D = list(T[:6]) + [jnp.maximum(T[6], A[9]), jnp.maximum(T[7], A[8]),
                       jnp.maximum(T[8], A[7]), jnp.maximum(T[9], A[6]),
                       A[5], A[4], A[3], A[2], A[1], A[0]]
    for i, j in _MERGE16:
        hi = jnp.maximum(D[i], D[j])
        lo = jnp.minimum(D[i], D[j])
        D[i], D[j] = hi, lo
    return D[:10]


def _merge_bot(B, A):
    D = list(B[:6]) + [jnp.minimum(B[6], A[6]), jnp.minimum(B[7], A[7]),
                       jnp.minimum(B[8], A[8]), jnp.minimum(B[9], A[9]),
                       A[10], A[11], A[12], A[13], A[14], A[15]]
    for i, j in _MERGE16:
        lo = jnp.minimum(D[i], D[j])
        hi = jnp.maximum(D[i], D[j])
        D[i], D[j] = lo, hi
    return D[:10]


def _make_sc_kernel(rows, n, num_cores=NUM_CORES):
    num_workers = num_cores * NUM_SUBCORES
    rows_per_w = rows // num_workers
    tiles = rows_per_w // TILE
    groups = n // GROUP

    mesh = plsc.VectorSubcoreMesh(
        core_axis_name="c", subcore_axis_name="s",
        num_cores=num_cores, num_subcores=NUM_SUBCORES)

    @functools.partial(
        pl.kernel,
        mesh=mesh,
        out_type=jax.ShapeDtypeStruct((rows,), jnp.float32),
        scratch_types=[
            pltpu.VMEM((TILE * n,), jnp.float32),
            pltpu.VMEM((TILE * n,), jnp.float32),
            pltpu.VMEM((rows_per_w,), jnp.float32),
            pltpu.SemaphoreType.DMA,
            pltpu.SemaphoreType.DMA,
        ],
        compiler_params=pltpu.CompilerParams(
            use_tc_tiling_on_sc=False, needs_layout_passes=False),
    )
    def k(x_hbm, out_hbm, buf_a, buf_b, out_v, sem_a, sem_b):
        wid = lax.axis_index("s") * NUM_CORES + lax.axis_index("c")
        row0 = wid * rows_per_w
        lanes = lax.iota(jnp.int32, LANES)
        lanebase = lanes * n
        rot = lanes * 17
        neg = jnp.full((LANES,), -jnp.inf, jnp.float32)
        pos = jnp.full((LANES,), jnp.inf, jnp.float32)

        SPLIT = (n - 255 - (GROUP - 1)) // GROUP

        def src(t):
            return x_hbm.at[pl.ds((row0 + t * TILE) * n, TILE * n)]

        def process(buf_v, t):
            def consume(T, Bo, A):
                A = _sort16_desc(A)
                return _merge_top(T, A), _merge_bot(Bo, A)

            def group_body_a(g, carry):
                T = list(carry[:KMAX])
                Bo = list(carry[KMAX:2 * KMAX])
                iv = carry[2 * KMAX]
                A = [plsc.load_gather(buf_v, [iv + kk])
                     for kk in range(GROUP)]
                T, Bo = consume(T, Bo, A)
                return tuple(T) + tuple(Bo) + (iv + GROUP,)

            def group_body_b(g, carry):
                T = list(carry[:KMAX])
                Bo = list(carry[KMAX:2 * KMAX])
                iv = carry[2 * KMAX]
                A = [plsc.load_gather(
                        buf_v, [lanebase + ((iv + kk) & (n - 1))])
                     for kk in range(GROUP)]
                T, Bo = consume(T, Bo, A)
                return tuple(T) + tuple(Bo) + (iv + GROUP,)

            init = (neg,) * KMAX + (pos,) * KMAX + (lanebase + rot,)
            mid = lax.fori_loop(0, SPLIT, group_body_a, init)
            init_b = mid[:2 * KMAX] + (rot + SPLIT * GROUP,)
            fin = lax.fori_loop(SPLIT, groups, group_body_b, init_b)
            top_sum = fin[0]
            for j in range(1, KMAX):
                top_sum = top_sum + fin[j]
            bot_sum = fin[KMAX]
            for j in range(1, KMIN):
                bot_sum = bot_sum + fin[KMAX + j]
            res = (top_sum / KMAX + bot_sum / KMIN) * jnp.float32(0.5)
            out_v[pl.ds(t * TILE, TILE)] = res

        npairs = tiles // 2
        pltpu.async_copy(src(0), buf_a, sem_a)

        def pair_body(tp, _):
            t0 = 2 * tp
            pltpu.make_async_copy(src(t0), buf_a, sem_a).wait()
            pltpu.async_copy(src(t0 + 1), buf_b, sem_b)
            process(buf_a, t0)
            pltpu.make_async_copy(src(t0 + 1), buf_b, sem_b).wait()

            @pl.when(tp + 1 < npairs)
            def _prefetch_next():
                pltpu.async_copy(src(t0 + 2), buf_a, sem_a)

            process(buf_b, t0 + 1)
            return 0

        lax.fori_loop(0, npairs, pair_body, 0)
        pltpu.sync_copy(out_v, out_hbm.at[pl.ds(row0, rows_per_w)])

    return k


ROWS_PER_BLOCK = 256


def _tc_body(x_ref, o_ref):
    x = x_ref[...]
    R, N = x.shape
    col = lax.broadcasted_iota(jnp.int32, (R, N), 1)
    neg = jnp.float32(-jnp.inf)
    pos = jnp.float32(jnp.inf)

    xt = x
    top_sum = jnp.zeros((R, 1), jnp.float32)
    for _ in range(KMAX):
        m = jnp.max(xt, axis=1, keepdims=True)
        top_sum = top_sum + m
        idx = jnp.min(jnp.where(xt == m, col, N), axis=1, keepdims=True)
        xt = jnp.where(col == idx, neg, xt)

    xb = x
    bot_sum = jnp.zeros((R, 1), jnp.float32)
    for _ in range(KMIN):
        m = jnp.min(xb, axis=1, keepdims=True)
        bot_sum = bot_sum + m
        idx = jnp.min(jnp.where(xb == m, col, N), axis=1, keepdims=True)
        xb = jnp.where(col == idx, pos, xb)

    o_ref[...] = (top_sum / KMAX + bot_sum / KMIN) * 0.5


def _tc_kernel(x, n):
    rows = x.shape[0]
    R = ROWS_PER_BLOCK
    out = pl.pallas_call(
        _tc_body,
        grid=(rows // R,),
        in_specs=[pl.BlockSpec((R, n), lambda i: (i, 0))],
        out_specs=pl.BlockSpec((R, 1), lambda i: (i, 0)),
        out_shape=jax.ShapeDtypeStruct((rows, 1), jnp.float32),
    )(x)
    return out.reshape(rows)


def kernel(input):
    B, C, H, W = input.shape
    n = H * W
    rows = B * C
    x = input.reshape(rows * n)
    out = _make_sc_kernel(rows, n)(x)
    return out.reshape(B, C)


# speedup vs baseline: 1.7305x; 1.1189x over previous
import functools

import jax
import jax.numpy as jnp
from jax import lax
from jax.experimental import pallas as pl
from jax.experimental.pallas import tpu as pltpu
from jax.experimental.pallas import tpu_sc as plsc

KMAX = 10
KMIN = 10

NUM_CORES = 2
NUM_SUBCORES = 16
LANES = 16
TILE = 16
GROUP = 16


def _batcher(num):
    def oe_merge(lo, nn, r):
        step = r * 2
        if step < nn:
            yield from oe_merge(lo, nn, step)
            yield from oe_merge(lo + r, nn, step)
            for i in range(lo + r, lo + nn - r, step):
                yield (i, i + r)
        else:
            yield (lo, lo + r)

    def srt(lo, nn):
        if nn > 1:
            m = nn // 2
            yield from srt(lo, m)
            yield from srt(lo + m, m)
            yield from oe_merge(lo, nn, 1)

    return list(srt(0, num))


_SORT16 = _batcher(GROUP)


def _sort16_desc(v):
    v = list(v)
    for i, j in _SORT16:
        hi = jnp.maximum(v[i], v[j])
        lo = jnp.minimum(v[i], v[j])
        v[i], v[j] = hi, lo
    return v


def _pruned_merge16_ces():
    ces = []
    for d in (8, 4, 2, 1):
        for j in range(16):
            if (j % (2 * d)) < d:
                ces.append((j, j + d))
    needed = set(range(10))
    keep = []
    for i, j in reversed(ces):
        if i in needed or j in needed:
            keep.append((i, j))
            needed.add(i)
            needed.add(j)
    keep.reverse()
    return keep


_MERGE16 = _pruned_merge16_ces()


def _merge_top(T, A):
    D = list(T[:6]) + [jnp.maximum(T[6], A[9]), jnp.maximum(T[7], A[8]),
                       jnp.maximum(T[8], A[7]), jnp.maximum(T[9], A[6]),
                       A[5], A[4], A[3], A[2], A[1], A[0]]
    for i, j in _MERGE16:
        hi = jnp.maximum(D[i], D[j])
        lo = jnp.minimum(D[i], D[j])
        D[i], D[j] = hi, lo
    return D[:10]


def _merge_bot(B, A):
    D = list(B[:6]) + [jnp.minimum(B[6], A[6]), jnp.minimum(B[7], A[7]),
                       jnp.minimum(B[8], A[8]), jnp.minimum(B[9], A[9]),
                       A[10], A[11], A[12], A[13], A[14], A[15]]
    for i, j in _MERGE16:
        lo = jnp.minimum(D[i], D[j])
        hi = jnp.maximum(D[i], D[j])
        D[i], D[j] = lo, hi
    return D[:10]


def _make_sc_kernel(rows, n, num_cores=NUM_CORES):
    num_workers = num_cores * NUM_SUBCORES
    rows_per_w = rows // num_workers
    tiles = rows_per_w // TILE
    groups = n // GROUP

    mesh = plsc.VectorSubcoreMesh(
        core_axis_name="c", subcore_axis_name="s",
        num_cores=num_cores, num_subcores=NUM_SUBCORES)

    @functools.partial(
        pl.kernel,
        mesh=mesh,
        out_type=jax.ShapeDtypeStruct((rows,), jnp.float32),
        scratch_types=[
            pltpu.VMEM((TILE, n), jnp.float32),
            pltpu.VMEM((TILE, n), jnp.float32),
            pltpu.VMEM((rows_per_w,), jnp.float32),
            pltpu.SemaphoreType.DMA,
            pltpu.SemaphoreType.DMA,
        ],
        compiler_params=pltpu.CompilerParams(
            use_tc_tiling_on_sc=False, needs_layout_passes=False),
    )
    def k(x_hbm, out_hbm, buf_a, buf_b, out_v, sem_a, sem_b):
        wid = lax.axis_index("s") * NUM_CORES + lax.axis_index("c")
        row0 = wid * rows_per_w
        lanes = lax.iota(jnp.int32, LANES)
        lanebase = lanes * n
        rot = lanes * 17
        neg = jnp.full((LANES,), -jnp.inf, jnp.float32)
        pos = jnp.full((LANES,), jnp.inf, jnp.float32)

        SPLIT = (n - 255 - (GROUP - 1)) // GROUP

        def src(t):
            return x_hbm.at[pl.ds(row0 + t * TILE, TILE)]

        def process(buf_v, t):
            def consume(T, Bo, A):
                A = _sort16_desc(A)
                return _merge_top(T, A), _merge_bot(Bo, A)

            def group_body_a(g, carry):
                T = list(carry[:KMAX])
                Bo = list(carry[KMAX:2 * KMAX])
                iv = carry[2 * KMAX]
                A = [plsc.load_gather(buf_v, [lanes, iv + kk])
                     for kk in range(GROUP)]
                T, Bo = consume(T, Bo, A)
                return tuple(T) + tuple(Bo) + (iv + GROUP,)

            def group_body_b(g, carry):
                T = list(carry[:KMAX])
                Bo = list(carry[KMAX:2 * KMAX])
                iv = carry[2 * KMAX]
                A = [plsc.load_gather(buf_v, [lanes, (iv + kk) & (n - 1)])
                     for kk in range(GROUP)]
                T, Bo = consume(T, Bo, A)
                return tuple(T) + tuple(Bo) + (iv + GROUP,)

            init = (neg,) * KMAX + (pos,) * KMAX + (rot,)
            mid = lax.fori_loop(0, SPLIT, group_body_a, init)
            init_b = mid[:2 * KMAX] + (rot + SPLIT * GROUP,)
            fin = lax.fori_loop(SPLIT, groups, group_body_b, init_b)
            top_sum = fin[0]
            for j in range(1, KMAX):
                top_sum = top_sum + fin[j]
            bot_sum = fin[KMAX]
            for j in range(1, KMIN):
                bot_sum = bot_sum + fin[KMAX + j]
            res = (top_sum / KMAX + bot_sum / KMIN) * jnp.float32(0.5)
            out_v[pl.ds(t * TILE, TILE)] = res

        npairs = tiles // 2
        pltpu.async_copy(src(0), buf_a, sem_a)

        def pair_body(tp, _):
            t0 = 2 * tp
            pltpu.make_async_copy(src(t0), buf_a, sem_a).wait()
            pltpu.async_copy(src(t0 + 1), buf_b, sem_b)
            process(buf_a, t0)
            pltpu.make_async_copy(src(t0 + 1), buf_b, sem_b).wait()

            @pl.when(tp + 1 < npairs)
            def _prefetch_next():
                pltpu.async_copy(src(t0 + 2), buf_a, sem_a)

            process(buf_b, t0 + 1)
            return 0

        lax.fori_loop(0, npairs, pair_body, 0)
        pltpu.sync_copy(out_v, out_hbm.at[pl.ds(row0, rows_per_w)])

    return k


ROWS_PER_BLOCK = 256


def _tc_body(x_ref, o_ref):
    x = x_ref[...]
    R, N = x.shape
    col = lax.broadcasted_iota(jnp.int32, (R, N), 1)
    neg = jnp.float32(-jnp.inf)
    pos = jnp.float32(jnp.inf)

    xt = x
    top_sum = jnp.zeros((R, 1), jnp.float32)
    for _ in range(KMAX):
        m = jnp.max(xt, axis=1, keepdims=True)
        top_sum = top_sum + m
        idx = jnp.min(jnp.where(xt == m, col, N), axis=1, keepdims=True)
        xt = jnp.where(col == idx, neg, xt)

    xb = x
    bot_sum = jnp.zeros((R, 1), jnp.float32)
    for _ in range(KMIN):
        m = jnp.min(xb, axis=1, keepdims=True)
        bot_sum = bot_sum + m
        idx = jnp.min(jnp.where(xb == m, col, N), axis=1, keepdims=True)
        xb = jnp.where(col == idx, pos, xb)

    o_ref[...] = (top_sum / KMAX + bot_sum / KMIN) * 0.5


def _tc_kernel(x, n):
    rows = x.shape[0]
    R = ROWS_PER_BLOCK
    out = pl.pallas_call(
        _tc_body,
        grid=(rows // R,),
        in_specs=[pl.BlockSpec((R, n), lambda i: (i, 0))],
        out_specs=pl.BlockSpec((R, 1), lambda i: (i, 0)),
        out_shape=jax.ShapeDtypeStruct((rows, 1), jnp.float32),
    )(x)
    return out.reshape(rows)


def kernel(input):
    B, C, H, W = input.shape
    n = H * W
    rows = B * C
    x = input.reshape(rows, n)
    out = _make_sc_kernel(rows, n)(x)
    return out.reshape(B, C)


# speedup vs baseline: 1.7311x; 1.0004x over previous
import functools

import jax
import jax.numpy as jnp
from jax import lax
from jax.experimental import pallas as pl
from jax.experimental.pallas import tpu as pltpu
from jax.experimental.pallas import tpu_sc as plsc

KMAX = 10
KMIN = 10

NUM_CORES = 2
NUM_SUBCORES = 16
LANES = 16
TILE = 16
GROUP = 16


def _batcher(num):
    def oe_merge(lo, nn, r):
        step = r * 2
        if step < nn:
            yield from oe_merge(lo, nn, step)
            yield from oe_merge(lo + r, nn, step)
            for i in range(lo + r, lo + nn - r, step):
                yield (i, i + r)
        else:
            yield (lo, lo + r)

    def srt(lo, nn):
        if nn > 1:
            m = nn // 2
            yield from srt(lo, m)
            yield from srt(lo + m, m)
            yield from oe_merge(lo, nn, 1)

    return list(srt(0, num))


_SORT16 = _batcher(GROUP)


def _sort16_desc(v):
    v = list(v)
    for i, j in _SORT16:
        hi = jnp.maximum(v[i], v[j])
        lo = jnp.minimum(v[i], v[j])
        v[i], v[j] = hi, lo
    return v


def _pruned_merge16_ces():
    ces = []
    for d in (8, 4, 2, 1):
        for j in range(16):
            if (j % (2 * d)) < d:
                ces.append((j, j + d))
    needed = set(range(10))
    keep = []
    for i, j in reversed(ces):
        if i in needed or j in needed:
            keep.append((i, j))
            needed.add(i)
            needed.add(j)
    keep.reverse()
    return keep


_MERGE16 = _pruned_merge16_ces()


def _merge_top(T, A):
    D = list(T[:6]) + [jnp.maximum(T[6], A[9]), jnp.maximum(T[7], A[8]),
                       jnp.maximum(T[8], A[7]), jnp.maximum(T[9], A[6]),
                       A[5], A[4], A[3], A[2], A[1], A[0]]
    for i, j in _MERGE16:
        hi = jnp.maximum(D[i], D[j])
        lo = jnp.minimum(D[i], D[j])
        D[i], D[j] = hi, lo
    return D[:10]


def _merge_bot(B, A):
    D = list(B[:6]) + [jnp.minimum(B[6], A[6]), jnp.minimum(B[7], A[7]),
                       jnp.minimum(B[8], A[8]), jnp.minimum(B[9], A[9]),
                       A[10], A[11], A[12], A[13], A[14], A[15]]
    for i, j in _MERGE16:
        lo = jnp.minimum(D[i], D[j])
        hi = jnp.maximum(D[i], D[j])
        D[i], D[j] = lo, hi
    return D[:10]


def _make_sc_kernel(rows, n, num_cores=NUM_CORES):
    num_workers = num_cores * NUM_SUBCORES
    rows_per_w = rows // num_workers
    tiles = rows_per_w // TILE
    groups = n // GROUP

    mesh = plsc.VectorSubcoreMesh(
        core_axis_name="c", subcore_axis_name="s",
        num_cores=num_cores, num_subcores=NUM_SUBCORES)

    @functools.partial(
        pl.kernel,
        mesh=mesh,
        out_type=jax.ShapeDtypeStruct((rows,), jnp.float32),
        scratch_types=[
            pltpu.VMEM((TILE, n), jnp.float32),
            pltpu.VMEM((TILE, n), jnp.float32),
            pltpu.VMEM((rows_per_w,), jnp.float32),
            pltpu.SemaphoreType.DMA,
            pltpu.SemaphoreType.DMA,
        ],
        compiler_params=pltpu.CompilerParams(
            use_tc_tiling_on_sc=False, needs_layout_passes=False),
    )
    def k(x_hbm, out_hbm, buf_a, buf_b, out_v, sem_a, sem_b):
        wid = lax.axis_index("s") * NUM_CORES + lax.axis_index("c")
        row0 = wid * rows_per_w
        lanes = lax.iota(jnp.int32, LANES)
        rot = lanes * 17
        neg = jnp.full((LANES,), -jnp.inf, jnp.float32)
        pos = jnp.full((LANES,), jnp.inf, jnp.float32)

        SPLIT = (n - 255 - (GROUP - 1)) // GROUP

        def src(t):
            return x_hbm.at[pl.ds(row0 + t * TILE, TILE)]

        def process(buf_v, t):
            def consume(T, Bo, A):
                A = _sort16_desc(A)
                return _merge_top(T, A), _merge_bot(Bo, A)

            def group_body_a(g, carry):
                T = list(carry[:KMAX])
                Bo = list(carry[KMAX:2 * KMAX])
                iv = carry[2 * KMAX]
                A = [plsc.load_gather(buf_v, [lanes, iv + kk])
                     for kk in range(GROUP)]
                T, Bo = consume(T, Bo, A)
                return tuple(T) + tuple(Bo) + (iv + GROUP,)

            def group_body_b(g, carry):
                T = list(carry[:KMAX])
                Bo = list(carry[KMAX:2 * KMAX])
                iv = carry[2 * KMAX]
                A = [plsc.load_gather(buf_v, [lanes, (iv + kk) & (n - 1)])
                     for kk in range(GROUP)]
                T, Bo = consume(T, Bo, A)
                return tuple(T) + tuple(Bo) + (iv + GROUP,)

            init = (neg,) * KMAX + (pos,) * KMAX + (rot,)
            mid = lax.fori_loop(0, SPLIT, group_body_a, init)
            init_b = mid[:2 * KMAX] + (rot + SPLIT * GROUP,)
            fin = lax.fori_loop(SPLIT, groups, group_body_b, init_b)
            top_sum = fin[0]
            for j in range(1, KMAX):
                top_sum = top_sum + fin[j]
            bot_sum = fin[KMAX]
            for j in range(1, KMIN):
                bot_sum = bot_sum + fin[KMAX + j]
            res = (top_sum / KMAX + bot_sum / KMIN) * jnp.float32(0.5)
            out_v[pl.ds(t * TILE, TILE)] = res

        npairs = tiles // 2
        pltpu.async_copy(src(0), buf_a, sem_a)

        def pair_body(tp, _):
            t0 = 2 * tp
            pltpu.make_async_copy(src(t0), buf_a, sem_a).wait()
            pltpu.async_copy(src(t0 + 1), buf_b, sem_b)
            process(buf_a, t0)
            pltpu.make_async_copy(src(t0 + 1), buf_b, sem_b).wait()

            @pl.when(tp + 1 < npairs)
            def _prefetch_next():
                pltpu.async_copy(src(t0 + 2), buf_a, sem_a)

            process(buf_b, t0 + 1)
            return 0

        lax.fori_loop(0, npairs, pair_body, 0)
        pltpu.sync_copy(out_v, out_hbm.at[pl.ds(row0, rows_per_w)])

    return k


ROWS_PER_BLOCK = 256


def _tc_body(x_ref, o_ref):
    x = x_ref[...]
    R, N = x.shape
    col = lax.broadcasted_iota(jnp.int32, (R, N), 1)
    neg = jnp.float32(-jnp.inf)
    pos = jnp.float32(jnp.inf)

    xt = x
    top_sum = jnp.zeros((R, 1), jnp.float32)
    for _ in range(KMAX):
        m = jnp.max(xt, axis=1, keepdims=True)
        top_sum = top_sum + m
        idx = jnp.min(jnp.where(xt == m, col, N), axis=1, keepdims=True)
        xt = jnp.where(col == idx, neg, xt)

    xb = x
    bot_sum = jnp.zeros((R, 1), jnp.float32)
    for _ in range(KMIN):
        m = jnp.min(xb, axis=1, keepdims=True)
        bot_sum = bot_sum + m
        idx = jnp.min(jnp.where(xb == m, col, N), axis=1, keepdims=True)
        xb = jnp.where(col == idx, pos, xb)

    o_ref[...] = (top_sum / KMAX + bot_sum / KMIN) * 0.5


def _tc_kernel(x, n):
    rows = x.shape[0]
    R = ROWS_PER_BLOCK
    out = pl.pallas_call(
        _tc_body,
        grid=(rows // R,),
        in_specs=[pl.BlockSpec((R, n), lambda i: (i, 0))],
        out_specs=pl.BlockSpec((R, 1), lambda i: (i, 0)),
        out_shape=jax.ShapeDtypeStruct((rows, 1), jnp.float32),
    )(x)
    return out.reshape(rows)


def kernel(input):
    B, C, H, W = input.shape
    n = H * W
    rows = B * C
    x = input.reshape(rows, n)
    out = _make_sc_kernel(rows, n)(x)
    return out.reshape(B, C)


# speedup vs baseline: 1.7327x; 1.0010x over previous
import functools

import jax
import jax.numpy as jnp
from jax import lax
from jax.experimental import pallas as pl
from jax.experimental.pallas import tpu as pltpu
from jax.experimental.pallas import tpu_sc as plsc

KMAX = 10
KMIN = 10

NUM_CORES = 2
NUM_SUBCORES = 16
LANES = 16
TILE = 16
GROUP = 16


def _batcher(num):
    def oe_merge(lo, nn, r):
        step = r * 2
        if step < nn:
            yield from oe_merge(lo, nn, step)
            yield from oe_merge(lo + r, nn, step)
            for i in range(lo + r, lo + nn - r, step):
                yield (i, i + r)
        else:
            yield (lo, lo + r)

    def srt(lo, nn):
        if nn > 1:
            m = nn // 2
            yield from srt(lo, m)
            yield from srt(lo + m, m)
            yield from oe_merge(lo, nn, 1)

    return list(srt(0, num))


_SORT16 = _batcher(GROUP)


def _sort16_desc(v):
    v = list(v)
    for i, j in _SORT16:
        hi = jnp.maximum(v[i], v[j])
        lo = jnp.minimum(v[i], v[j])
        v[i], v[j] = hi, lo
    return v


def _pruned_merge16_ces():
    ces = []
    for d in (8, 4, 2, 1):
        for j in range(16):
            if (j % (2 * d)) < d:
                ces.append((j, j + d))
    needed = set(range(10))
    keep = []
    for i, j in reversed(ces):
        if i in needed or j in needed:
            keep.append((i, j))
            needed.add(i)
            needed.add(j)
    keep.reverse()
    return keep


_MERGE16 = _pruned_merge16_ces()


def _merge_top(T, A):
    D = list(T[:6]) + [jnp.maximum(T[6], A[9]), jnp.maximum(T[7], A[8]),
                       jnp.maximum(T[8], A[7]), jnp.maximum(T[9], A[6]),
                       A[5], A[4], A[3], A[2], A[1], A[0]]
    for i, j in _MERGE16:
        hi = jnp.maximum(D[i], D[j])
        lo = jnp.minimum(D[i], D[j])
        D[i], D[j] = hi, lo
    return D[:10]


def _merge_bot(B, A):
    D = list(B[:6]) + [jnp.minimum(B[6], A[6]), jnp.minimum(B[7], A[7]),
                       jnp.minimum(B[8], A[8]), jnp.minimum(B[9], A[9]),
                       A[10], A[11], A[12], A[13], A[14], A[15]]
    for i, j in _MERGE16:
        lo = jnp.minimum(D[i], D[j])
        hi = jnp.maximum(D[i], D[j])
        D[i], D[j] = lo, hi
    return D[:10]


def _make_sc_kernel(rows, n, num_cores=NUM_CORES):
    num_workers = num_cores * NUM_SUBCORES
    rows_per_w = rows // num_workers
    tiles = rows_per_w // TILE
    groups = n // GROUP

    mesh = plsc.VectorSubcoreMesh(
        core_axis_name="c", subcore_axis_name="s",
        num_cores=num_cores, num_subcores=NUM_SUBCORES)

    @functools.partial(
        pl.kernel,
        mesh=mesh,
        out_type=jax.ShapeDtypeStruct((rows,), jnp.float32),
        scratch_types=[
            pltpu.VMEM((TILE, n), jnp.float32),
            pltpu.VMEM((TILE, n), jnp.float32),
            pltpu.VMEM((rows_per_w,), jnp.float32),
            pltpu.SemaphoreType.DMA,
            pltpu.SemaphoreType.DMA,
        ],
        compiler_params=pltpu.CompilerParams(
            use_tc_tiling_on_sc=False, needs_layout_passes=False),
    )
    def k(x_hbm, out_hbm, buf_a, buf_b, out_v, sem_a, sem_b):
        wid = lax.axis_index("s") * NUM_CORES + lax.axis_index("c")
        row0 = wid * rows_per_w
        lanes = lax.iota(jnp.int32, LANES)
        rot = lanes * 17
        neg = jnp.full((LANES,), -jnp.inf, jnp.float32)
        pos = jnp.full((LANES,), jnp.inf, jnp.float32)

        SPLIT = (n - 255 - (GROUP - 1)) // GROUP

        def src(t):
            return x_hbm.at[pl.ds(row0 + t * TILE, TILE)]

        def process(buf_v, t):
            def consume(T, Bo, A):
                A = _sort16_desc(A)
                return _merge_top(T, A), _merge_bot(Bo, A)

            def group_body_a(g, carry):
                T = list(carry[:KMAX])
                Bo = list(carry[KMAX:2 * KMAX])
                iv = carry[2 * KMAX]
                A = [plsc.load_gather(buf_v, [lanes, iv + kk])
                     for kk in range(GROUP)]
                T, Bo = consume(T, Bo, A)
                return tuple(T) + tuple(Bo) + (iv + GROUP,)

            def group_body_b(g, carry):
                T = list(carry[:KMAX])
                Bo = list(carry[KMAX:2 * KMAX])
                iv = carry[2 * KMAX]
                A = [plsc.load_gather(buf_v, [lanes, (iv + kk) & (n - 1)])
                     for kk in range(GROUP)]
                T, Bo = consume(T, Bo, A)
                return tuple(T) + tuple(Bo) + (iv + GROUP,)

            init = (neg,) * KMAX + (pos,) * KMAX + (rot,)
            mid = lax.fori_loop(0, SPLIT, group_body_a, init)
            init_b = mid[:2 * KMAX] + (rot + SPLIT * GROUP,)
            fin = lax.fori_loop(SPLIT, groups, group_body_b, init_b)
            top_sum = fin[0]
            for j in range(1, KMAX):
                top_sum = top_sum + fin[j]
            bot_sum = fin[KMAX]
            for j in range(1, KMIN):
                bot_sum = bot_sum + fin[KMAX + j]
            res = (top_sum / KMAX + bot_sum / KMIN) * jnp.float32(0.5)
            out_v[pl.ds(t * TILE, TILE)] = res

        npairs = tiles // 2
        pltpu.async_copy(src(0), buf_a, sem_a)

        def pair_body(tp, _):
            t0 = 2 * tp
            pltpu.make_async_copy(src(t0), buf_a, sem_a).wait()
            pltpu.async_copy(src(t0 + 1), buf_b, sem_b)
            process(buf_a, t0)
            pltpu.make_async_copy(src(t0 + 1), buf_b, sem_b).wait()

            @pl.when(tp + 1 < npairs)
            def _prefetch_next():
                pltpu.async_copy(src(t0 + 2), buf_a, sem_a)

            process(buf_b, t0 + 1)
            return 0

        lax.fori_loop(0, npairs, pair_body, 0)
        pltpu.sync_copy(out_v, out_hbm.at[pl.ds(row0, rows_per_w)])

    return k


def kernel(input):
    B, C, H, W = input.shape
    n = H * W
    rows = B * C
    x = input.reshape(rows, n)
    out = _make_sc_kernel(rows, n)(x)
    return out.reshape(B, C)
